# baseline (speedup 1.0000x reference)
import functools

import jax
import jax.numpy as jnp
from jax import lax
from jax.experimental import pallas as pl
from jax.experimental.pallas import tpu as pltpu
from jax.experimental.pallas import tpu_sc as plsc

NC = 2
NS = 16
NW = NC * NS
LB = 128


def _edge_pass(nt_rows, roww, np_rows, pb):
  nmsg = (roww - 16) // 16
  rows_per_tile = np_rows // NS
  nchunks = rows_per_tile // LB
  mesh = plsc.VectorSubcoreMesh(core_axis_name="c", subcore_axis_name="s")

  @functools.partial(
      pl.kernel,
      out_type=jax.ShapeDtypeStruct((NC, np_rows, roww), jnp.float32),
      mesh=mesh,
      compiler_params=pltpu.CompilerParams(
          use_tc_tiling_on_sc=False, needs_layout_passes=False),
      scratch_types=[
          pltpu.VMEM_SHARED((np_rows, roww), jnp.float32),
          pltpu.VMEM((pb, LB), jnp.int32),
          pltpu.VMEM((pb, LB), jnp.int32),
          pltpu.VMEM((LB, roww), jnp.float32),
          pltpu.VMEM((LB, 16), jnp.float32),
          pltpu.VMEM((LB, roww), jnp.float32),
          pltpu.VMEM((LB, 16), jnp.float32),
          pltpu.SemaphoreType.DMA,
          pltpu.SemaphoreType.DMA,
      ],
  )
  def edge_pass(tbl_s, tbl_d, src2d, dst2d, out_acc, acc, idx_s, idx_d,
                srows, drows, msg, wscr, sem1, sem2):
    c = lax.axis_index("c")
    s = lax.axis_index("s")
    w = s * NC + c

    zero16 = jnp.zeros((16,), jnp.float32)

    def zrow(i, carry):
      for k in range(roww // 16):
        msg[i, pl.ds(16 * k, 16)] = zero16
      return carry

    lax.fori_loop(0, LB, zrow, 0)
    for t in range(nchunks):
      pltpu.sync_copy(msg, acc.at[pl.ds((s * nchunks + t) * LB, LB)])
    plsc.subcore_barrier()

    pltpu.sync_copy(src2d.at[pl.ds(w * pb, pb)], idx_s)
    pltpu.sync_copy(dst2d.at[pl.ds(w * pb, pb)], idx_d)

    lane = lax.iota(jnp.int32, 16)
    half = lane // 8

    def batch(j, carry):
      cp1 = pltpu.async_copy(tbl_s.at[idx_s.at[j]], srows, sem1)
      cp2 = pltpu.async_copy(tbl_d.at[idx_d.at[j]], drows, sem2)
      cp1.wait()
      cp2.wait()

      def edge(e, ecarry):
        a = srows[e, pl.ds(0, 16)]
        b = drows[e, pl.ds(0, 16)]
        t0 = a + b
        wv = jnp.exp(jnp.maximum(t0, 0.2 * t0))
        msg[e, pl.ds(0, 16)] = wv
        if nmsg == 1:
          hi = srows[e, pl.ds(16, 16)]
          msg[e, pl.ds(16, 16)] = wv * hi
        else:
          wscr[e, pl.ds(0, 16)] = wv
          for i in range(nmsg):
            wb = plsc.load_gather(wscr.at[e], [half + 2 * i])
            hi = srows[e, pl.ds(16 + 16 * i, 16)]
            msg[e, pl.ds(16 + 16 * i, 16)] = wb * hi
        return ecarry

      lax.fori_loop(0, LB, edge, 0, unroll=4)
      pltpu.sync_copy(msg, acc.at[idx_d.at[j]], add=True)
      return carry

    lax.fori_loop(0, pb, batch, 0)
    plsc.subcore_barrier()

    for t in range(nchunks):
      r0 = (s * nchunks + t) * LB
      pltpu.sync_copy(acc.at[pl.ds(r0, LB)], out_acc.at[c, pl.ds(r0, LB)])

  return edge_pass


def _tc_layer1(x, W1, A1s, A1d, n_block):
  n = x.shape[0]

  def body(x_ref, w_ref, as_ref, ad_ref, ts_ref, td_ref):
    h = jnp.dot(x_ref[...], w_ref[...], preferred_element_type=jnp.float32)
    asv = jnp.dot(h, as_ref[...], preferred_element_type=jnp.float32)
    adv = jnp.dot(h, ad_ref[...], preferred_element_type=jnp.float32)
    z = jnp.zeros((h.shape[0], 8), jnp.float32)
    ts_ref[...] = jnp.concatenate([asv, z, h], axis=1)
    td_ref[...] = jnp.concatenate([adv, z], axis=1)

  grid = n // n_block
  return pl.pallas_call(
      body,
      grid=(grid,),
      in_specs=[
          pl.BlockSpec((n_block, x.shape[1]), lambda i: (i, 0)),
          pl.BlockSpec(W1.shape, lambda i: (0, 0)),
          pl.BlockSpec(A1s.shape, lambda i: (0, 0)),
          pl.BlockSpec(A1d.shape, lambda i: (0, 0)),
      ],
      out_specs=[
          pl.BlockSpec((n_block, 80), lambda i: (i, 0)),
          pl.BlockSpec((n_block, 16), lambda i: (i, 0)),
      ],
      out_shape=[
          jax.ShapeDtypeStruct((n, 80), jnp.float32),
          jax.ShapeDtypeStruct((n, 16), jnp.float32),
      ],
  )(x, W1, A1s, A1d)


def _tc_layer2(acc1, b1, W2, a2sT, a2dT, n, n_block):
  np_rows, roww = acc1.shape[1], acc1.shape[2]

  def body(acc_ref, b1_ref, w2_ref, as_ref, ad_ref, ts_ref, td_ref):
    accsum = acc_ref[0] + acc_ref[1]
    denom = accsum[:, 0:8] + 1e-16
    msgf = accsum[:, 16:80].reshape(accsum.shape[0], 8, 8)
    out1 = (msgf / denom[:, :, None]).reshape(accsum.shape[0], 64)
    out1 = out1 + b1_ref[...]
    he = jnp.where(out1 > 0, out1, jnp.exp(jnp.minimum(out1, 0.0)) - 1.0)
    h2 = jnp.dot(he, w2_ref[...], preferred_element_type=jnp.float32)
    a2s = jnp.dot(h2, as_ref[...], preferred_element_type=jnp.float32)
    a2d = jnp.dot(h2, ad_ref[...], preferred_element_type=jnp.float32)
    ones = jnp.ones((1, 16), jnp.float32)
    ts_ref[...] = jnp.concatenate([a2s * ones, h2], axis=1)
    td_ref[...] = a2d * ones

  grid = n // n_block
  return pl.pallas_call(
      body,
      grid=(grid,),
      in_specs=[
          pl.BlockSpec((2, n_block, roww), lambda i: (0, i, 0)),
          pl.BlockSpec(b1.shape, lambda i: (0, 0)),
          pl.BlockSpec(W2.shape, lambda i: (0, 0)),
          pl.BlockSpec(a2sT.shape, lambda i: (0, 0)),
          pl.BlockSpec(a2dT.shape, lambda i: (0, 0)),
      ],
      out_specs=[
          pl.BlockSpec((n_block, 32), lambda i: (i, 0)),
          pl.BlockSpec((n_block, 16), lambda i: (i, 0)),
      ],
      out_shape=[
          jax.ShapeDtypeStruct((n, 32), jnp.float32),
          jax.ShapeDtypeStruct((n, 16), jnp.float32),
      ],
  )(acc1, b1, W2, a2sT, a2dT)


def _tc_final(acc2, b2, n, n_block):
  np_rows, roww = acc2.shape[1], acc2.shape[2]

  def body(acc_ref, b2_ref, out_ref):
    accsum = acc_ref[0] + acc_ref[1]
    denom = accsum[:, 0:1] + 1e-16
    out_ref[...] = accsum[:, 16:32] / denom + b2_ref[...]

  grid = n // n_block
  return pl.pallas_call(
      body,
      grid=(grid,),
      in_specs=[
          pl.BlockSpec((2, n_block, roww), lambda i: (0, i, 0)),
          pl.BlockSpec(b2.shape, lambda i: (0, 0)),
      ],
      out_specs=pl.BlockSpec((n_block, 16), lambda i: (i, 0)),
      out_shape=jax.ShapeDtypeStruct((n, 16), jnp.float32),
  )(acc2, b2)


def kernel(x, adj, W1, a1_src, a1_dst, b1, W2, a2_src, a2_dst, b2):
  n = x.shape[0]
  e = adj.shape[1]
  heads, hid = a1_src.shape
  out_f = W2.shape[1]

  pb = -(-e // (NW * LB))
  pb = ((pb + 7) // 8) * 8
  epad = pb * LB * NW - e
  dummy_dst = (n + 8 + jnp.arange(epad, dtype=adj.dtype) % 224)[None, :]
  dummy_src = jnp.full((1, epad), n, dtype=adj.dtype)
  adjp = jnp.concatenate(
      [adj, jnp.concatenate([dummy_src, dummy_dst], axis=0)], axis=1)
  src2d = adjp[0].reshape(NW * pb, LB)
  dst2d = adjp[1].reshape(NW * pb, LB)

  eye = jnp.eye(heads, dtype=jnp.float32)
  A1s = (a1_src[:, :, None] * eye[:, None, :]).reshape(heads * hid, heads)
  A1d = (a1_dst[:, :, None] * eye[:, None, :]).reshape(heads * hid, heads)

  np_rows = 10240
  nt = n + 8

  ts1, td1 = _tc_layer1(x, W1, A1s, A1d, n_block=2000)
  zs = jnp.zeros((nt - n, 80), jnp.float32)
  zd = jnp.zeros((nt - n, 16), jnp.float32)
  tbl1_s = jnp.concatenate([ts1, zs], axis=0)
  tbl1_d = jnp.concatenate([td1, zd], axis=0)
  acc1 = _edge_pass(nt, 80, np_rows, pb)(tbl1_s, tbl1_d, src2d, dst2d)

  ts2, td2 = _tc_layer2(acc1, b1.reshape(1, -1), W2,
                        a2_src.T, a2_dst.T, n, n_block=2000)
  tbl2_s = jnp.concatenate([ts2, jnp.zeros((nt - n, 32), jnp.float32)], axis=0)
  tbl2_d = jnp.concatenate([td2, jnp.zeros((nt - n, 16), jnp.float32)], axis=0)
  acc2 = _edge_pass(nt, 32, np_rows, pb)(tbl2_s, tbl2_d, src2d, dst2d)

  return _tc_final(acc2, b2.reshape(1, -1), n, n_block=2000)


# speedup vs baseline: 1.0296x; 1.0296x over previous
import functools

import jax
import jax.numpy as jnp
from jax import lax
from jax.experimental import pallas as pl
from jax.experimental.pallas import tpu as pltpu
from jax.experimental.pallas import tpu_sc as plsc

NC = 2
NS = 16
NW = NC * NS
LB = 128


def _edge_pass(nt_rows, roww, np_rows, pb):
  nmsg = (roww - 16) // 16
  rows_per_tile = np_rows // NS
  nchunks = rows_per_tile // LB
  mesh = plsc.VectorSubcoreMesh(core_axis_name="c", subcore_axis_name="s")

  @functools.partial(
      pl.kernel,
      out_type=jax.ShapeDtypeStruct((NC, np_rows, roww), jnp.float32),
      mesh=mesh,
      compiler_params=pltpu.CompilerParams(
          use_tc_tiling_on_sc=False, needs_layout_passes=False),
      scratch_types=[
          pltpu.VMEM_SHARED((np_rows, roww), jnp.float32),
          pltpu.VMEM((pb, LB), jnp.int32),
          pltpu.VMEM((pb, LB), jnp.int32),
          pltpu.VMEM((LB, roww), jnp.float32),
          pltpu.VMEM((LB, 16), jnp.float32),
          pltpu.VMEM((LB, roww), jnp.float32),
          pltpu.VMEM((LB, 16), jnp.float32),
          pltpu.SemaphoreType.DMA,
          pltpu.SemaphoreType.DMA,
      ],
  )
  def edge_pass(tbl_s, tbl_d, src2d, dst2d, out_acc, acc, idx_s, idx_d,
                srows, drows, msg, wscr, sem1, sem2):
    c = lax.axis_index("c")
    s = lax.axis_index("s")
    w = s * NC + c

    zero16 = jnp.zeros((16,), jnp.float32)

    def zrow(i, carry):
      for k in range(roww // 16):
        msg[i, pl.ds(16 * k, 16)] = zero16
      return carry

    lax.fori_loop(0, LB, zrow, 0)
    for t in range(nchunks):
      pltpu.sync_copy(msg, acc.at[pl.ds((s * nchunks + t) * LB, LB)])
    plsc.subcore_barrier()

    pltpu.sync_copy(src2d.at[pl.ds(w * pb, pb)], idx_s)
    pltpu.sync_copy(dst2d.at[pl.ds(w * pb, pb)], idx_d)

    lane = lax.iota(jnp.int32, 16)
    half = lane // 8

    def batch(j, carry):
      cp1 = pltpu.async_copy(tbl_s.at[idx_s.at[j]], srows, sem1)
      cp2 = pltpu.async_copy(tbl_d.at[idx_d.at[j]], drows, sem2)
      cp1.wait()
      cp2.wait()

      def edge(e, ecarry):
        a = srows[e, pl.ds(0, 16)]
        b = drows[e, pl.ds(0, 16)]
        t0 = a + b
        wv = jnp.exp(jnp.maximum(t0, 0.2 * t0))
        msg[e, pl.ds(0, 16)] = wv
        if nmsg == 1:
          hi = srows[e, pl.ds(16, 16)]
          msg[e, pl.ds(16, 16)] = wv * hi
        else:
          wscr[e, pl.ds(0, 16)] = wv
          for i in range(nmsg):
            wb = plsc.load_gather(wscr.at[e], [half + 2 * i])
            hi = srows[e, pl.ds(16 + 16 * i, 16)]
            msg[e, pl.ds(16 + 16 * i, 16)] = wb * hi
        return ecarry

      lax.fori_loop(0, LB, edge, 0)
      pltpu.sync_copy(msg, acc.at[idx_d.at[j]], add=True)
      return carry

    lax.fori_loop(0, pb, batch, 0)
    plsc.subcore_barrier()

    for t in range(nchunks):
      r0 = (s * nchunks + t) * LB
      pltpu.sync_copy(acc.at[pl.ds(r0, LB)], out_acc.at[c, pl.ds(r0, LB)])

  return edge_pass


def _tc_layer1(x, W1, A1s, A1d, n_block):
  n = x.shape[0]

  def body(x_ref, w_ref, as_ref, ad_ref, ts_ref, td_ref):
    h = jnp.dot(x_ref[...], w_ref[...], preferred_element_type=jnp.float32)
    asv = jnp.dot(h, as_ref[...], preferred_element_type=jnp.float32)
    adv = jnp.dot(h, ad_ref[...], preferred_element_type=jnp.float32)
    z = jnp.zeros((h.shape[0], 8), jnp.float32)
    ts_ref[...] = jnp.concatenate([asv, z, h], axis=1)
    td_ref[...] = jnp.concatenate([adv, z], axis=1)

  grid = n // n_block
  return pl.pallas_call(
      body,
      grid=(grid,),
      in_specs=[
          pl.BlockSpec((n_block, x.shape[1]), lambda i: (i, 0)),
          pl.BlockSpec(W1.shape, lambda i: (0, 0)),
          pl.BlockSpec(A1s.shape, lambda i: (0, 0)),
          pl.BlockSpec(A1d.shape, lambda i: (0, 0)),
      ],
      out_specs=[
          pl.BlockSpec((n_block, 80), lambda i: (i, 0)),
          pl.BlockSpec((n_block, 16), lambda i: (i, 0)),
      ],
      out_shape=[
          jax.ShapeDtypeStruct((n, 80), jnp.float32),
          jax.ShapeDtypeStruct((n, 16), jnp.float32),
      ],
  )(x, W1, A1s, A1d)


def _tc_layer2(acc1, b1, W2, a2sT, a2dT, n, n_block):
  np_rows, roww = acc1.shape[1], acc1.shape[2]

  def body(acc_ref, b1_ref, w2_ref, as_ref, ad_ref, ts_ref, td_ref):
    accsum = acc_ref[0] + acc_ref[1]
    denom = accsum[:, 0:8] + 1e-16
    msgf = accsum[:, 16:80].reshape(accsum.shape[0], 8, 8)
    out1 = (msgf / denom[:, :, None]).reshape(accsum.shape[0], 64)
    out1 = out1 + b1_ref[...]
    he = jnp.where(out1 > 0, out1, jnp.exp(jnp.minimum(out1, 0.0)) - 1.0)
    h2 = jnp.dot(he, w2_ref[...], preferred_element_type=jnp.float32)
    a2s = jnp.dot(h2, as_ref[...], preferred_element_type=jnp.float32)
    a2d = jnp.dot(h2, ad_ref[...], preferred_element_type=jnp.float32)
    ones = jnp.ones((1, 16), jnp.float32)
    ts_ref[...] = jnp.concatenate([a2s * ones, h2], axis=1)
    td_ref[...] = a2d * ones

  grid = n // n_block
  return pl.pallas_call(
      body,
      grid=(grid,),
      in_specs=[
          pl.BlockSpec((2, n_block, roww), lambda i: (0, i, 0)),
          pl.BlockSpec(b1.shape, lambda i: (0, 0)),
          pl.BlockSpec(W2.shape, lambda i: (0, 0)),
          pl.BlockSpec(a2sT.shape, lambda i: (0, 0)),
          pl.BlockSpec(a2dT.shape, lambda i: (0, 0)),
      ],
      out_specs=[
          pl.BlockSpec((n_block, 32), lambda i: (i, 0)),
          pl.BlockSpec((n_block, 16), lambda i: (i, 0)),
      ],
      out_shape=[
          jax.ShapeDtypeStruct((n, 32), jnp.float32),
          jax.ShapeDtypeStruct((n, 16), jnp.float32),
      ],
  )(acc1, b1, W2, a2sT, a2dT)


def _tc_final(acc2, b2, n, n_block):
  np_rows, roww = acc2.shape[1], acc2.shape[2]

  def body(acc_ref, b2_ref, out_ref):
    accsum = acc_ref[0] + acc_ref[1]
    denom = accsum[:, 0:1] + 1e-16
    out_ref[...] = accsum[:, 16:32] / denom + b2_ref[...]

  grid = n // n_block
  return pl.pallas_call(
      body,
      grid=(grid,),
      in_specs=[
          pl.BlockSpec((2, n_block, roww), lambda i: (0, i, 0)),
          pl.BlockSpec(b2.shape, lambda i: (0, 0)),
      ],
      out_specs=pl.BlockSpec((n_block, 16), lambda i: (i, 0)),
      out_shape=jax.ShapeDtypeStruct((n, 16), jnp.float32),
  )(acc2, b2)


def kernel(x, adj, W1, a1_src, a1_dst, b1, W2, a2_src, a2_dst, b2):
  n = x.shape[0]
  e = adj.shape[1]
  heads, hid = a1_src.shape
  out_f = W2.shape[1]

  pb = -(-e // (NW * LB))
  pb = ((pb + 7) // 8) * 8
  epad = pb * LB * NW - e
  dummy_dst = (n + 8 + jnp.arange(epad, dtype=adj.dtype) % 224)[None, :]
  dummy_src = jnp.full((1, epad), n, dtype=adj.dtype)
  adjp = jnp.concatenate(
      [adj, jnp.concatenate([dummy_src, dummy_dst], axis=0)], axis=1)
  src2d = adjp[0].reshape(NW * pb, LB)
  dst2d = adjp[1].reshape(NW * pb, LB)

  eye = jnp.eye(heads, dtype=jnp.float32)
  A1s = (a1_src[:, :, None] * eye[:, None, :]).reshape(heads * hid, heads)
  A1d = (a1_dst[:, :, None] * eye[:, None, :]).reshape(heads * hid, heads)

  np_rows = 10240
  nt = n + 8

  ts1, td1 = _tc_layer1(x, W1, A1s, A1d, n_block=2000)
  zs = jnp.zeros((nt - n, 80), jnp.float32)
  zd = jnp.zeros((nt - n, 16), jnp.float32)
  tbl1_s = jnp.concatenate([ts1, zs], axis=0)
  tbl1_d = jnp.concatenate([td1, zd], axis=0)
  acc1 = _edge_pass(nt, 80, np_rows, pb)(tbl1_s, tbl1_d, src2d, dst2d)

  ts2, td2 = _tc_layer2(acc1, b1.reshape(1, -1), W2,
                        a2_src.T, a2_dst.T, n, n_block=2000)
  tbl2_s = jnp.concatenate([ts2, jnp.zeros((nt - n, 32), jnp.float32)], axis=0)
  tbl2_d = jnp.concatenate([td2, jnp.zeros((nt - n, 16), jnp.float32)], axis=0)
  acc2 = _edge_pass(nt, 32, np_rows, pb)(tbl2_s, tbl2_d, src2d, dst2d)

  return _tc_final(acc2, b2.reshape(1, -1), n, n_block=2000)


# speedup vs baseline: 1.0538x; 1.0235x over previous
import functools

import jax
import jax.numpy as jnp
from jax import lax
from jax.experimental import pallas as pl
from jax.experimental.pallas import tpu as pltpu
from jax.experimental.pallas import tpu_sc as plsc

NC = 2
NS = 16
NW = NC * NS
LB = 128


def _edge_pass(nt_rows, roww, np_rows, pb):
  nmsg = (roww - 16) // 16
  rows_per_tile = np_rows // NS
  nchunks = rows_per_tile // LB
  nt8 = nt_rows * 8
  mesh = plsc.VectorSubcoreMesh(core_axis_name="c", subcore_axis_name="s")

  @functools.partial(
      pl.kernel,
      out_type=jax.ShapeDtypeStruct((NC, np_rows, roww), jnp.float32),
      mesh=mesh,
      compiler_params=pltpu.CompilerParams(
          use_tc_tiling_on_sc=False, needs_layout_passes=False),
      scratch_types=[
          pltpu.VMEM_SHARED((np_rows, roww), jnp.float32),
          pltpu.VMEM((pb, LB), jnp.int32),
          pltpu.VMEM((pb, LB), jnp.int32),
          pltpu.VMEM((LB, roww), jnp.float32),
          pltpu.VMEM((LB, 16), jnp.float32),
          pltpu.VMEM((LB, roww), jnp.float32),
          pltpu.VMEM((LB, 16), jnp.float32),
          pltpu.SemaphoreType.DMA,
          pltpu.SemaphoreType.DMA,
      ],
  )
  def edge_pass(tbl_s, tbl_d, src2d, dst2d, out_acc, acc, idx_s, idx_d,
                srows, drows, msg, wscr, sem1, sem2):
    c = lax.axis_index("c")
    s = lax.axis_index("s")
    w = s * NC + c

    zero16 = jnp.zeros((16,), jnp.float32)

    def zrow(i, carry):
      for k in range(roww // 16):
        msg[i, pl.ds(16 * k, 16)] = zero16
      return carry

    lax.fori_loop(0, LB, zrow, 0)
    for t in range(nchunks):
      pltpu.sync_copy(msg, acc.at[pl.ds((s * nchunks + t) * LB, LB)])
    plsc.subcore_barrier()

    pltpu.sync_copy(src2d.at[pl.ds(w * pb, pb)], idx_s)
    pltpu.sync_copy(dst2d.at[pl.ds(w * pb, pb)], idx_d)

    lane = lax.iota(jnp.int32, 16)
    half = lane // 8

    lane7 = lane & 7

    def batch(j, carry):
      cp1 = pltpu.async_copy(tbl_s.at[idx_s.at[j]], srows, sem1)
      cp2 = pltpu.async_copy(tbl_d.at[idx_d.at[j]], drows, sem2)
      cp1.wait()
      cp2.wait()

      def group(g, gcarry):
        for k in range(16):
          e = g * 16 + k
          a = srows[e, pl.ds(0, 16)]
          b = drows[e, pl.ds(0, 16)]
          t0 = a + b
          wv = jnp.exp(jnp.maximum(t0, 0.2 * t0))
          msg[e, pl.ds(0, 16)] = wv
          if nmsg == 1:
            hi = srows[e, pl.ds(16, 16)]
            msg[e, pl.ds(16, 16)] = wv * hi
          else:
            wscr[e, pl.ds(0, 16)] = wv
            for i in range(nmsg):
              wb = plsc.load_gather(wscr.at[e], [half + 2 * i])
              hi = srows[e, pl.ds(16 + 16 * i, 16)]
              msg[e, pl.ds(16 + 16 * i, 16)] = wb * hi
        return gcarry

      lax.fori_loop(0, LB // 16, group, 0)
      pltpu.sync_copy(msg, acc.at[idx_d.at[j]], add=True)
      return carry

    lax.fori_loop(0, pb, batch, 0)
    plsc.subcore_barrier()

    for t in range(nchunks):
      r0 = (s * nchunks + t) * LB
      pltpu.sync_copy(acc.at[pl.ds(r0, LB)], out_acc.at[c, pl.ds(r0, LB)])

  return edge_pass


def _tc_layer1(x, W1, A1s, A1d, n_block):
  n = x.shape[0]

  def body(x_ref, w_ref, as_ref, ad_ref, ts_ref, td_ref):
    h = jnp.dot(x_ref[...], w_ref[...], preferred_element_type=jnp.float32)
    asv = jnp.dot(h, as_ref[...], preferred_element_type=jnp.float32)
    adv = jnp.dot(h, ad_ref[...], preferred_element_type=jnp.float32)
    z = jnp.zeros((h.shape[0], 8), jnp.float32)
    ts_ref[...] = jnp.concatenate([asv, z, h], axis=1)
    td_ref[...] = jnp.concatenate([adv, z], axis=1)

  grid = n // n_block
  return pl.pallas_call(
      body,
      grid=(grid,),
      in_specs=[
          pl.BlockSpec((n_block, x.shape[1]), lambda i: (i, 0)),
          pl.BlockSpec(W1.shape, lambda i: (0, 0)),
          pl.BlockSpec(A1s.shape, lambda i: (0, 0)),
          pl.BlockSpec(A1d.shape, lambda i: (0, 0)),
      ],
      out_specs=[
          pl.BlockSpec((n_block, 80), lambda i: (i, 0)),
          pl.BlockSpec((n_block, 16), lambda i: (i, 0)),
      ],
      out_shape=[
          jax.ShapeDtypeStruct((n, 80), jnp.float32),
          jax.ShapeDtypeStruct((n, 16), jnp.float32),
      ],
  )(x, W1, A1s, A1d)


def _tc_layer2(acc1, b1, W2, a2sT, a2dT, n, n_block):
  np_rows, roww = acc1.shape[1], acc1.shape[2]

  def body(acc_ref, b1_ref, w2_ref, as_ref, ad_ref, ts_ref, td_ref):
    accsum = acc_ref[0] + acc_ref[1]
    denom = accsum[:, 0:8] + 1e-16
    msgf = accsum[:, 16:80].reshape(accsum.shape[0], 8, 8)
    out1 = (msgf / denom[:, :, None]).reshape(accsum.shape[0], 64)
    out1 = out1 + b1_ref[...]
    he = jnp.where(out1 > 0, out1, jnp.exp(jnp.minimum(out1, 0.0)) - 1.0)
    h2 = jnp.dot(he, w2_ref[...], preferred_element_type=jnp.float32)
    a2s = jnp.dot(h2, as_ref[...], preferred_element_type=jnp.float32)
    a2d = jnp.dot(h2, ad_ref[...], preferred_element_type=jnp.float32)
    ts_ref[...] = jnp.concatenate([a2s * jnp.ones((1, 16), jnp.float32), h2],
                                  axis=1)
    td_ref[...] = a2d * jnp.ones((1, 16), jnp.float32)

  grid = n // n_block
  return pl.pallas_call(
      body,
      grid=(grid,),
      in_specs=[
          pl.BlockSpec((2, n_block, roww), lambda i: (0, i, 0)),
          pl.BlockSpec(b1.shape, lambda i: (0, 0)),
          pl.BlockSpec(W2.shape, lambda i: (0, 0)),
          pl.BlockSpec(a2sT.shape, lambda i: (0, 0)),
          pl.BlockSpec(a2dT.shape, lambda i: (0, 0)),
      ],
      out_specs=[
          pl.BlockSpec((n_block, 32), lambda i: (i, 0)),
          pl.BlockSpec((n_block, 16), lambda i: (i, 0)),
      ],
      out_shape=[
          jax.ShapeDtypeStruct((n, 32), jnp.float32),
          jax.ShapeDtypeStruct((n, 16), jnp.float32),
      ],
  )(acc1, b1, W2, a2sT, a2dT)


def _tc_final(acc2, b2, n, n_block):
  np_rows, roww = acc2.shape[1], acc2.shape[2]

  def body(acc_ref, b2_ref, out_ref):
    accsum = acc_ref[0] + acc_ref[1]
    denom = accsum[:, 0:1] + 1e-16
    out_ref[...] = accsum[:, 16:32] / denom + b2_ref[...]

  grid = n // n_block
  return pl.pallas_call(
      body,
      grid=(grid,),
      in_specs=[
          pl.BlockSpec((2, n_block, roww), lambda i: (0, i, 0)),
          pl.BlockSpec(b2.shape, lambda i: (0, 0)),
      ],
      out_specs=pl.BlockSpec((n_block, 16), lambda i: (i, 0)),
      out_shape=jax.ShapeDtypeStruct((n, 16), jnp.float32),
  )(acc2, b2)


def kernel(x, adj, W1, a1_src, a1_dst, b1, W2, a2_src, a2_dst, b2):
  n = x.shape[0]
  e = adj.shape[1]
  heads, hid = a1_src.shape
  out_f = W2.shape[1]

  pb = -(-e // (NW * LB))
  pb = ((pb + 7) // 8) * 8
  epad = pb * LB * NW - e
  dummy_dst = (n + 8 + jnp.arange(epad, dtype=adj.dtype) % 224)[None, :]
  dummy_src = jnp.full((1, epad), n, dtype=adj.dtype)
  adjp = jnp.concatenate(
      [adj, jnp.concatenate([dummy_src, dummy_dst], axis=0)], axis=1)
  src2d = adjp[0].reshape(NW * pb, LB)
  dst2d = adjp[1].reshape(NW * pb, LB)

  eye = jnp.eye(heads, dtype=jnp.float32)
  A1s = (a1_src[:, :, None] * eye[:, None, :]).reshape(heads * hid, heads)
  A1d = (a1_dst[:, :, None] * eye[:, None, :]).reshape(heads * hid, heads)

  np_rows = 10240
  nt = n + 8

  ts1, td1 = _tc_layer1(x, W1, A1s, A1d, n_block=2000)
  zs = jnp.zeros((nt - n, 80), jnp.float32)
  zd = jnp.zeros((nt - n, 16), jnp.float32)
  tbl1_s = jnp.concatenate([ts1, zs], axis=0)
  tbl1_d = jnp.concatenate([td1, zd], axis=0)
  acc1 = _edge_pass(nt, 80, np_rows, pb)(tbl1_s, tbl1_d, src2d, dst2d)

  ts2, td2 = _tc_layer2(acc1, b1.reshape(1, -1), W2,
                        a2_src.T, a2_dst.T, n, n_block=2000)
  tbl2_s = jnp.concatenate([ts2, jnp.zeros((nt - n, 32), jnp.float32)], axis=0)
  tbl2_d = jnp.concatenate([td2, jnp.zeros((nt - n, 16), jnp.float32)], axis=0)
  acc2 = _edge_pass(nt, 32, np_rows, pb)(tbl2_s, tbl2_d, src2d, dst2d)

  return _tc_final(acc2, b2.reshape(1, -1), n, n_block=2000)


# speedup vs baseline: 1.6822x; 1.5963x over previous
import functools

import jax
import jax.numpy as jnp
from jax import lax
from jax.experimental import pallas as pl
from jax.experimental.pallas import tpu as pltpu
from jax.experimental.pallas import tpu_sc as plsc

NC = 2
NS = 16
NW = NC * NS
LB = 128


def _edge_pass(nt_rows, roww, np_rows, pb):
  nmsg = (roww - 16) // 16
  rows_per_tile = np_rows // NS
  nchunks = rows_per_tile // LB
  nt8 = nt_rows * 8
  mesh = plsc.VectorSubcoreMesh(core_axis_name="c", subcore_axis_name="s")

  @functools.partial(
      pl.kernel,
      out_type=jax.ShapeDtypeStruct((NC, np_rows, roww), jnp.float32),
      mesh=mesh,
      compiler_params=pltpu.CompilerParams(
          use_tc_tiling_on_sc=False, needs_layout_passes=False),
      scratch_types=[
          pltpu.VMEM_SHARED((np_rows, roww), jnp.float32),
          pltpu.VMEM((pb, LB), jnp.int32),
          pltpu.VMEM((pb, LB), jnp.int32),
          pltpu.VMEM((LB, roww), jnp.float32),
          pltpu.VMEM((LB, roww), jnp.float32),
          pltpu.VMEM((LB, 16), jnp.float32),
          pltpu.VMEM((LB, 16), jnp.float32),
          pltpu.VMEM((LB, roww), jnp.float32),
          pltpu.VMEM((LB, 16), jnp.float32),
          pltpu.SemaphoreType.DMA,
          pltpu.SemaphoreType.DMA,
          pltpu.SemaphoreType.DMA,
          pltpu.SemaphoreType.DMA,
      ],
  )
  def edge_pass(tbl_s, tbl_d, src2d, dst2d, out_acc, acc, idx_s, idx_d,
                srows0, srows1, drows0, drows1, msg, wscr,
                sems0, sems1, semd0, semd1):
    c = lax.axis_index("c")
    s = lax.axis_index("s")
    w = s * NC + c

    zero16 = jnp.zeros((16,), jnp.float32)

    def zrow(i, carry):
      for k in range(roww // 16):
        msg[i, pl.ds(16 * k, 16)] = zero16
      return carry

    lax.fori_loop(0, LB, zrow, 0)
    for t in range(nchunks):
      pltpu.sync_copy(msg, acc.at[pl.ds((s * nchunks + t) * LB, LB)])
    plsc.subcore_barrier()

    pltpu.sync_copy(src2d.at[pl.ds(w * pb, pb)], idx_s)
    pltpu.sync_copy(dst2d.at[pl.ds(w * pb, pb)], idx_d)

    lane = lax.iota(jnp.int32, 16)
    half = lane // 8

    lane7 = lane & 7

    def start(j, sbuf, dbuf, ssem, dsem):
      pltpu.async_copy(tbl_s.at[idx_s.at[j]], sbuf, ssem)
      pltpu.async_copy(tbl_d.at[idx_d.at[j]], dbuf, dsem)

    def wait(j, sbuf, dbuf, ssem, dsem):
      pltpu.make_async_copy(tbl_s.at[idx_s.at[j]], sbuf, ssem).wait()
      pltpu.make_async_copy(tbl_d.at[idx_d.at[j]], dbuf, dsem).wait()

    def process(j, sbuf, dbuf):
      def group(g, gcarry):
        for k in range(16):
          e = g * 16 + k
          a = sbuf[e, pl.ds(0, 16)]
          b = dbuf[e, pl.ds(0, 16)]
          t0 = a + b
          wv = jnp.exp(jnp.maximum(t0, 0.2 * t0))
          msg[e, pl.ds(0, 16)] = wv
          if nmsg == 1:
            hi = sbuf[e, pl.ds(16, 16)]
            msg[e, pl.ds(16, 16)] = wv * hi
          else:
            wscr[e, pl.ds(0, 16)] = wv
            for i in range(nmsg):
              wb = plsc.load_gather(wscr.at[e], [half + 2 * i])
              hi = sbuf[e, pl.ds(16 + 16 * i, 16)]
              msg[e, pl.ds(16 + 16 * i, 16)] = wb * hi
        return gcarry

      lax.fori_loop(0, LB // 16, group, 0)
      pltpu.sync_copy(msg, acc.at[idx_d.at[j]], add=True)

    start(0, srows0, drows0, sems0, semd0)

    def pair(i, carry):
      j0 = 2 * i
      j1 = j0 + 1
      start(j1, srows1, drows1, sems1, semd1)
      wait(j0, srows0, drows0, sems0, semd0)
      process(j0, srows0, drows0)
      start(jnp.minimum(j0 + 2, pb - 1), srows0, drows0, sems0, semd0)
      wait(j1, srows1, drows1, sems1, semd1)
      process(j1, srows1, drows1)
      return carry

    lax.fori_loop(0, pb // 2, pair, 0)
    wait(pb - 1, srows0, drows0, sems0, semd0)
    plsc.subcore_barrier()

    for t in range(nchunks):
      r0 = (s * nchunks + t) * LB
      pltpu.sync_copy(acc.at[pl.ds(r0, LB)], out_acc.at[c, pl.ds(r0, LB)])

  return edge_pass


def _tc_layer1(x, W1, A1s, A1d, n_block):
  n = x.shape[0]

  def body(x_ref, w_ref, as_ref, ad_ref, ts_ref, td_ref):
    h = jnp.dot(x_ref[...], w_ref[...], preferred_element_type=jnp.float32)
    asv = jnp.dot(h, as_ref[...], preferred_element_type=jnp.float32)
    adv = jnp.dot(h, ad_ref[...], preferred_element_type=jnp.float32)
    z = jnp.zeros((h.shape[0], 8), jnp.float32)
    ts_ref[...] = jnp.concatenate([asv, z, h], axis=1)
    td_ref[...] = jnp.concatenate([adv, z], axis=1)

  grid = n // n_block
  return pl.pallas_call(
      body,
      grid=(grid,),
      in_specs=[
          pl.BlockSpec((n_block, x.shape[1]), lambda i: (i, 0)),
          pl.BlockSpec(W1.shape, lambda i: (0, 0)),
          pl.BlockSpec(A1s.shape, lambda i: (0, 0)),
          pl.BlockSpec(A1d.shape, lambda i: (0, 0)),
      ],
      out_specs=[
          pl.BlockSpec((n_block, 80), lambda i: (i, 0)),
          pl.BlockSpec((n_block, 16), lambda i: (i, 0)),
      ],
      out_shape=[
          jax.ShapeDtypeStruct((n, 80), jnp.float32),
          jax.ShapeDtypeStruct((n, 16), jnp.float32),
      ],
  )(x, W1, A1s, A1d)


def _tc_layer2(acc1, b1, W2, a2sT, a2dT, n, n_block):
  np_rows, roww = acc1.shape[1], acc1.shape[2]

  def body(acc_ref, b1_ref, w2_ref, as_ref, ad_ref, ts_ref, td_ref):
    accsum = acc_ref[0] + acc_ref[1]
    denom = accsum[:, 0:8] + 1e-16
    msgf = accsum[:, 16:80].reshape(accsum.shape[0], 8, 8)
    out1 = (msgf / denom[:, :, None]).reshape(accsum.shape[0], 64)
    out1 = out1 + b1_ref[...]
    he = jnp.where(out1 > 0, out1, jnp.exp(jnp.minimum(out1, 0.0)) - 1.0)
    h2 = jnp.dot(he, w2_ref[...], preferred_element_type=jnp.float32)
    a2s = jnp.dot(h2, as_ref[...], preferred_element_type=jnp.float32)
    a2d = jnp.dot(h2, ad_ref[...], preferred_element_type=jnp.float32)
    ts_ref[...] = jnp.concatenate([a2s * jnp.ones((1, 16), jnp.float32), h2],
                                  axis=1)
    td_ref[...] = a2d * jnp.ones((1, 16), jnp.float32)

  grid = n // n_block
  return pl.pallas_call(
      body,
      grid=(grid,),
      in_specs=[
          pl.BlockSpec((2, n_block, roww), lambda i: (0, i, 0)),
          pl.BlockSpec(b1.shape, lambda i: (0, 0)),
          pl.BlockSpec(W2.shape, lambda i: (0, 0)),
          pl.BlockSpec(a2sT.shape, lambda i: (0, 0)),
          pl.BlockSpec(a2dT.shape, lambda i: (0, 0)),
      ],
      out_specs=[
          pl.BlockSpec((n_block, 32), lambda i: (i, 0)),
          pl.BlockSpec((n_block, 16), lambda i: (i, 0)),
      ],
      out_shape=[
          jax.ShapeDtypeStruct((n, 32), jnp.float32),
          jax.ShapeDtypeStruct((n, 16), jnp.float32),
      ],
  )(acc1, b1, W2, a2sT, a2dT)


def _tc_final(acc2, b2, n, n_block):
  np_rows, roww = acc2.shape[1], acc2.shape[2]

  def body(acc_ref, b2_ref, out_ref):
    accsum = acc_ref[0] + acc_ref[1]
    denom = accsum[:, 0:1] + 1e-16
    out_ref[...] = accsum[:, 16:32] / denom + b2_ref[...]

  grid = n // n_block
  return pl.pallas_call(
      body,
      grid=(grid,),
      in_specs=[
          pl.BlockSpec((2, n_block, roww), lambda i: (0, i, 0)),
          pl.BlockSpec(b2.shape, lambda i: (0, 0)),
      ],
      out_specs=pl.BlockSpec((n_block, 16), lambda i: (i, 0)),
      out_shape=jax.ShapeDtypeStruct((n, 16), jnp.float32),
  )(acc2, b2)


def kernel(x, adj, W1, a1_src, a1_dst, b1, W2, a2_src, a2_dst, b2):
  n = x.shape[0]
  e = adj.shape[1]
  heads, hid = a1_src.shape
  out_f = W2.shape[1]

  pb = -(-e // (NW * LB))
  pb = ((pb + 7) // 8) * 8
  epad = pb * LB * NW - e
  dummy_dst = (n + 8 + jnp.arange(epad, dtype=adj.dtype) % 224)[None, :]
  dummy_src = jnp.full((1, epad), n, dtype=adj.dtype)
  adjp = jnp.concatenate(
      [adj, jnp.concatenate([dummy_src, dummy_dst], axis=0)], axis=1)
  src2d = adjp[0].reshape(NW * pb, LB)
  dst2d = adjp[1].reshape(NW * pb, LB)

  eye = jnp.eye(heads, dtype=jnp.float32)
  A1s = (a1_src[:, :, None] * eye[:, None, :]).reshape(heads * hid, heads)
  A1d = (a1_dst[:, :, None] * eye[:, None, :]).reshape(heads * hid, heads)

  np_rows = 10240
  nt = n + 8

  ts1, td1 = _tc_layer1(x, W1, A1s, A1d, n_block=2000)
  zs = jnp.zeros((nt - n, 80), jnp.float32)
  zd = jnp.zeros((nt - n, 16), jnp.float32)
  tbl1_s = jnp.concatenate([ts1, zs], axis=0)
  tbl1_d = jnp.concatenate([td1, zd], axis=0)
  acc1 = _edge_pass(nt, 80, np_rows, pb)(tbl1_s, tbl1_d, src2d, dst2d)

  ts2, td2 = _tc_layer2(acc1, b1.reshape(1, -1), W2,
                        a2_src.T, a2_dst.T, n, n_block=2000)
  tbl2_s = jnp.concatenate([ts2, jnp.zeros((nt - n, 32), jnp.float32)], axis=0)
  tbl2_d = jnp.concatenate([td2, jnp.zeros((nt - n, 16), jnp.float32)], axis=0)
  acc2 = _edge_pass(nt, 32, np_rows, pb)(tbl2_s, tbl2_d, src2d, dst2d)

  return _tc_final(acc2, b2.reshape(1, -1), n, n_block=2000)


# speedup vs baseline: 1.9028x; 1.1312x over previous
import functools

import jax
import jax.numpy as jnp
from jax import lax
from jax.experimental import pallas as pl
from jax.experimental.pallas import tpu as pltpu
from jax.experimental.pallas import tpu_sc as plsc

NC = 2
NS = 16
NW = NC * NS
LB = 128


def _edge_pass(nt_rows, roww, np_rows, pb):
  nmsg = (roww - 16) // 16
  rows_per_tile = np_rows // NS
  nchunks = rows_per_tile // LB
  nt8 = nt_rows * 8
  mesh = plsc.VectorSubcoreMesh(core_axis_name="c", subcore_axis_name="s")

  @functools.partial(
      pl.kernel,
      out_type=jax.ShapeDtypeStruct((NC, np_rows, roww), jnp.float32),
      mesh=mesh,
      compiler_params=pltpu.CompilerParams(
          use_tc_tiling_on_sc=False, needs_layout_passes=False),
      scratch_types=[
          pltpu.VMEM_SHARED((np_rows, roww), jnp.float32),
          pltpu.VMEM((pb, LB), jnp.int32),
          pltpu.VMEM((pb, LB), jnp.int32),
          pltpu.VMEM((LB, roww), jnp.float32),
          pltpu.VMEM((LB, roww), jnp.float32),
          pltpu.VMEM((LB, 16), jnp.float32),
          pltpu.VMEM((LB, 16), jnp.float32),
          pltpu.VMEM((LB, roww), jnp.float32),
          pltpu.VMEM((LB, 16), jnp.float32),
          pltpu.SemaphoreType.DMA,
          pltpu.SemaphoreType.DMA,
          pltpu.SemaphoreType.DMA,
          pltpu.SemaphoreType.DMA,
      ],
  )
  def edge_pass(tbl_s, tbl_d, src2d, dst2d, out_acc, acc, idx_s, idx_d,
                srows0, srows1, drows0, drows1, msg, wscr,
                sems0, sems1, semd0, semd1):
    c = lax.axis_index("c")
    s = lax.axis_index("s")
    w = s * NC + c

    zero16 = jnp.zeros((16,), jnp.float32)

    def zrow(i, carry):
      for k in range(roww // 16):
        msg[i, pl.ds(16 * k, 16)] = zero16
      return carry

    lax.fori_loop(0, LB, zrow, 0)
    for t in range(nchunks):
      pltpu.sync_copy(msg, acc.at[pl.ds((s * nchunks + t) * LB, LB)])
    plsc.subcore_barrier()

    pltpu.sync_copy(src2d.at[pl.ds(w * pb, pb)], idx_s)
    pltpu.sync_copy(dst2d.at[pl.ds(w * pb, pb)], idx_d)

    lane = lax.iota(jnp.int32, 16)
    half = lane // 8

    lane7 = lane & 7

    def start(j, sbuf, dbuf, ssem, dsem):
      pltpu.async_copy(tbl_s.at[idx_s.at[j]], sbuf, ssem)
      pltpu.async_copy(tbl_d.at[idx_d.at[j]], dbuf, dsem)

    def wait(j, sbuf, dbuf, ssem, dsem):
      pltpu.make_async_copy(tbl_s.at[idx_s.at[j]], sbuf, ssem).wait()
      pltpu.make_async_copy(tbl_d.at[idx_d.at[j]], dbuf, dsem).wait()

    def process(j, sbuf, dbuf):
      def group(g, gcarry):
        for k in range(16):
          e = g * 16 + k
          a = sbuf[e, pl.ds(0, 16)]
          b = dbuf[e, pl.ds(0, 16)]
          t0 = a + b
          wv = jnp.exp(jnp.maximum(t0, 0.2 * t0))
          msg[e, pl.ds(0, 16)] = wv
          if nmsg == 1:
            hi = sbuf[e, pl.ds(16, 16)]
            msg[e, pl.ds(16, 16)] = wv * hi
          else:
            for i in range(nmsg):
              wb = wv.at[half + 2 * i].get(mode="promise_in_bounds")
              hi = sbuf[e, pl.ds(16 + 16 * i, 16)]
              msg[e, pl.ds(16 + 16 * i, 16)] = wb * hi
        return gcarry

      lax.fori_loop(0, LB // 16, group, 0)
      pltpu.sync_copy(msg, acc.at[idx_d.at[j]], add=True)

    start(0, srows0, drows0, sems0, semd0)

    def pair(i, carry):
      j0 = 2 * i
      j1 = j0 + 1
      start(j1, srows1, drows1, sems1, semd1)
      wait(j0, srows0, drows0, sems0, semd0)
      process(j0, srows0, drows0)
      start(jnp.minimum(j0 + 2, pb - 1), srows0, drows0, sems0, semd0)
      wait(j1, srows1, drows1, sems1, semd1)
      process(j1, srows1, drows1)
      return carry

    lax.fori_loop(0, pb // 2, pair, 0)
    wait(pb - 1, srows0, drows0, sems0, semd0)
    plsc.subcore_barrier()

    for t in range(nchunks):
      r0 = (s * nchunks + t) * LB
      pltpu.sync_copy(acc.at[pl.ds(r0, LB)], out_acc.at[c, pl.ds(r0, LB)])

  return edge_pass


def _tc_layer1(x, W1, A1s, A1d, n_block):
  n = x.shape[0]

  def body(x_ref, w_ref, as_ref, ad_ref, ts_ref, td_ref):
    h = jnp.dot(x_ref[...], w_ref[...], preferred_element_type=jnp.float32)
    asv = jnp.dot(h, as_ref[...], preferred_element_type=jnp.float32)
    adv = jnp.dot(h, ad_ref[...], preferred_element_type=jnp.float32)
    z = jnp.zeros((h.shape[0], 8), jnp.float32)
    ts_ref[...] = jnp.concatenate([asv, z, h], axis=1)
    td_ref[...] = jnp.concatenate([adv, z], axis=1)

  grid = n // n_block
  return pl.pallas_call(
      body,
      grid=(grid,),
      in_specs=[
          pl.BlockSpec((n_block, x.shape[1]), lambda i: (i, 0)),
          pl.BlockSpec(W1.shape, lambda i: (0, 0)),
          pl.BlockSpec(A1s.shape, lambda i: (0, 0)),
          pl.BlockSpec(A1d.shape, lambda i: (0, 0)),
      ],
      out_specs=[
          pl.BlockSpec((n_block, 80), lambda i: (i, 0)),
          pl.BlockSpec((n_block, 16), lambda i: (i, 0)),
      ],
      out_shape=[
          jax.ShapeDtypeStruct((n, 80), jnp.float32),
          jax.ShapeDtypeStruct((n, 16), jnp.float32),
      ],
  )(x, W1, A1s, A1d)


def _tc_layer2(acc1, b1, W2, a2sT, a2dT, n, n_block):
  np_rows, roww = acc1.shape[1], acc1.shape[2]

  def body(acc_ref, b1_ref, w2_ref, as_ref, ad_ref, ts_ref, td_ref):
    accsum = acc_ref[0] + acc_ref[1]
    denom = accsum[:, 0:8] + 1e-16
    msgf = accsum[:, 16:80].reshape(accsum.shape[0], 8, 8)
    out1 = (msgf / denom[:, :, None]).reshape(accsum.shape[0], 64)
    out1 = out1 + b1_ref[...]
    he = jnp.where(out1 > 0, out1, jnp.exp(jnp.minimum(out1, 0.0)) - 1.0)
    h2 = jnp.dot(he, w2_ref[...], preferred_element_type=jnp.float32)
    a2s = jnp.dot(h2, as_ref[...], preferred_element_type=jnp.float32)
    a2d = jnp.dot(h2, ad_ref[...], preferred_element_type=jnp.float32)
    ts_ref[...] = jnp.concatenate([a2s * jnp.ones((1, 16), jnp.float32), h2],
                                  axis=1)
    td_ref[...] = a2d * jnp.ones((1, 16), jnp.float32)

  grid = n // n_block
  return pl.pallas_call(
      body,
      grid=(grid,),
      in_specs=[
          pl.BlockSpec((2, n_block, roww), lambda i: (0, i, 0)),
          pl.BlockSpec(b1.shape, lambda i: (0, 0)),
          pl.BlockSpec(W2.shape, lambda i: (0, 0)),
          pl.BlockSpec(a2sT.shape, lambda i: (0, 0)),
          pl.BlockSpec(a2dT.shape, lambda i: (0, 0)),
      ],
      out_specs=[
          pl.BlockSpec((n_block, 32), lambda i: (i, 0)),
          pl.BlockSpec((n_block, 16), lambda i: (i, 0)),
      ],
      out_shape=[
          jax.ShapeDtypeStruct((n, 32), jnp.float32),
          jax.ShapeDtypeStruct((n, 16), jnp.float32),
      ],
  )(acc1, b1, W2, a2sT, a2dT)


def _tc_final(acc2, b2, n, n_block):
  np_rows, roww = acc2.shape[1], acc2.shape[2]

  def body(acc_ref, b2_ref, out_ref):
    accsum = acc_ref[0] + acc_ref[1]
    denom = accsum[:, 0:1] + 1e-16
    out_ref[...] = accsum[:, 16:32] / denom + b2_ref[...]

  grid = n // n_block
  return pl.pallas_call(
      body,
      grid=(grid,),
      in_specs=[
          pl.BlockSpec((2, n_block, roww), lambda i: (0, i, 0)),
          pl.BlockSpec(b2.shape, lambda i: (0, 0)),
      ],
      out_specs=pl.BlockSpec((n_block, 16), lambda i: (i, 0)),
      out_shape=jax.ShapeDtypeStruct((n, 16), jnp.float32),
  )(acc2, b2)


def kernel(x, adj, W1, a1_src, a1_dst, b1, W2, a2_src, a2_dst, b2):
  n = x.shape[0]
  e = adj.shape[1]
  heads, hid = a1_src.shape
  out_f = W2.shape[1]

  pb = -(-e // (NW * LB))
  pb = ((pb + 7) // 8) * 8
  epad = pb * LB * NW - e
  dummy_dst = (n + 8 + jnp.arange(epad, dtype=adj.dtype) % 224)[None, :]
  dummy_src = jnp.full((1, epad), n, dtype=adj.dtype)
  adjp = jnp.concatenate(
      [adj, jnp.concatenate([dummy_src, dummy_dst], axis=0)], axis=1)
  src2d = adjp[0].reshape(NW * pb, LB)
  dst2d = adjp[1].reshape(NW * pb, LB)

  eye = jnp.eye(heads, dtype=jnp.float32)
  A1s = (a1_src[:, :, None] * eye[:, None, :]).reshape(heads * hid, heads)
  A1d = (a1_dst[:, :, None] * eye[:, None, :]).reshape(heads * hid, heads)

  np_rows = 10240
  nt = n + 8

  ts1, td1 = _tc_layer1(x, W1, A1s, A1d, n_block=2000)
  zs = jnp.zeros((nt - n, 80), jnp.float32)
  zd = jnp.zeros((nt - n, 16), jnp.float32)
  tbl1_s = jnp.concatenate([ts1, zs], axis=0)
  tbl1_d = jnp.concatenate([td1, zd], axis=0)
  acc1 = _edge_pass(nt, 80, np_rows, pb)(tbl1_s, tbl1_d, src2d, dst2d)

  ts2, td2 = _tc_layer2(acc1, b1.reshape(1, -1), W2,
                        a2_src.T, a2_dst.T, n, n_block=2000)
  tbl2_s = jnp.concatenate([ts2, jnp.zeros((nt - n, 32), jnp.float32)], axis=0)
  tbl2_d = jnp.concatenate([td2, jnp.zeros((nt - n, 16), jnp.float32)], axis=0)
  acc2 = _edge_pass(nt, 32, np_rows, pb)(tbl2_s, tbl2_d, src2d, dst2d)

  return _tc_final(acc2, b2.reshape(1, -1), n, n_block=2000)
